# Initial kernel scaffold; baseline (speedup 1.0000x reference)
import functools

import jax
import jax.numpy as jnp
from jax import lax
from jax.experimental import pallas as pl
from jax.experimental.pallas import tpu as pltpu
from jax.experimental.pallas import tpu_sc as plsc

N_NODES = 10000
D_FEAT = 128
N_EDGES = 320000

NC = 2
NS = 16
EPT = N_EDGES // (NC * NS)
CHUNK = 80
NCHUNK = EPT // CHUNK
RPT = N_NODES // NS
ZROWS = 125
DEG_W = 16

_MESH = dict(core_axis_name="c", subcore_axis_name="s")


def _sc_degree(row):
    @functools.partial(
        pl.kernel,
        mesh=plsc.VectorSubcoreMesh(**_MESH),
        out_type=jax.ShapeDtypeStruct((NC, N_NODES, DEG_W), jnp.float32),
        scratch_types=[
            pltpu.VMEM((CHUNK,), jnp.int32),
            pltpu.VMEM((CHUNK, DEG_W), jnp.float32),
            pltpu.VMEM((ZROWS, DEG_W), jnp.float32),
            pltpu.VMEM_SHARED((N_NODES, DEG_W), jnp.float32),
        ],
    )
    def deg_kernel(row_hbm, out_hbm, idx_v, ones_v, zbuf_v, hist_sh):
        cid = lax.axis_index("c")
        sid = lax.axis_index("s")
        zero16 = jnp.zeros((16,), jnp.float32)
        one16 = jnp.ones((16,), jnp.float32)

        def fill_z(i, carry):
            zbuf_v[i, :] = zero16
            return carry

        lax.fori_loop(0, ZROWS, fill_z, 0)

        def fill_o(i, carry):
            ones_v[i, :] = one16
            return carry

        lax.fori_loop(0, CHUNK, fill_o, 0)

        row0 = sid * RPT
        for z in range(RPT // ZROWS):
            pltpu.sync_copy(zbuf_v, hist_sh.at[pl.ds(row0 + z * ZROWS, ZROWS), :])
        plsc.subcore_barrier()

        ebase = (cid * NS + sid) * EPT

        def body(c, carry):
            base = pl.multiple_of(ebase + c * CHUNK, 8)
            pltpu.sync_copy(row_hbm.at[pl.ds(base, CHUNK)], idx_v)
            pltpu.sync_copy(ones_v, hist_sh.at[idx_v], add=True)
            return carry

        lax.fori_loop(0, NCHUNK, body, 0)
        plsc.subcore_barrier()
        pltpu.sync_copy(hist_sh.at[pl.ds(row0, RPT), :],
                        out_hbm.at[cid, pl.ds(row0, RPT), :])

    return deg_kernel(row)


def _sc_propagate(row, col, g):
    @functools.partial(
        pl.kernel,
        mesh=plsc.VectorSubcoreMesh(**_MESH),
        out_type=jax.ShapeDtypeStruct((NC, N_NODES, D_FEAT), jnp.float32),
        scratch_types=[
            pltpu.VMEM((CHUNK,), jnp.int32),
            pltpu.VMEM((CHUNK,), jnp.int32),
            pltpu.VMEM((CHUNK, D_FEAT), jnp.float32),
            pltpu.VMEM((ZROWS, D_FEAT), jnp.float32),
            pltpu.VMEM_SHARED((N_NODES, D_FEAT), jnp.float32),
            pltpu.SemaphoreType.DMA,
        ],
    )
    def gs_kernel(row_hbm, col_hbm, g_hbm, out_hbm,
                  cidx_v, ridx_v, rows_v, zbuf_v, agg_sh, sem):
        cid = lax.axis_index("c")
        sid = lax.axis_index("s")
        zero16 = jnp.zeros((16,), jnp.float32)

        def fill_z(i, carry):
            for j in range(D_FEAT // 16):
                zbuf_v[i, pl.ds(j * 16, 16)] = zero16
            return carry

        lax.fori_loop(0, ZROWS, fill_z, 0)

        row0 = sid * RPT
        for z in range(RPT // ZROWS):
            pltpu.sync_copy(zbuf_v, agg_sh.at[pl.ds(row0 + z * ZROWS, ZROWS), :])
        plsc.subcore_barrier()

        ebase = (cid * NS + sid) * EPT

        def body(c, carry):
            base = pl.multiple_of(ebase + c * CHUNK, 8)
            pltpu.sync_copy(col_hbm.at[pl.ds(base, CHUNK)], cidx_v)
            pltpu.async_copy(g_hbm.at[cidx_v], rows_v, sem).wait()
            pltpu.sync_copy(row_hbm.at[pl.ds(base, CHUNK)], ridx_v)
            pltpu.sync_copy(rows_v, agg_sh.at[ridx_v], add=True)
            return carry

        lax.fori_loop(0, NCHUNK, body, 0)
        plsc.subcore_barrier()
        pltpu.sync_copy(agg_sh.at[pl.ds(row0, RPT), :],
                        out_hbm.at[cid, pl.ds(row0, RPT), :])

    return gs_kernel(row, col, g)


ROWS_BLK = 1000


def _dis_block(degp):
    deg = (degp[0] + degp[1])[:, 0:1]
    return jnp.where(deg > 0, 1.0 / jnp.sqrt(deg), 0.0)


def _tc_prescale(degp, x):
    def body(degp_ref, x_ref, g_ref):
        g_ref[...] = x_ref[...] * _dis_block(degp_ref[...])

    return pl.pallas_call(
        body,
        grid=(N_NODES // ROWS_BLK,),
        in_specs=[
            pl.BlockSpec((NC, ROWS_BLK, DEG_W), lambda i: (0, i, 0)),
            pl.BlockSpec((ROWS_BLK, D_FEAT), lambda i: (i, 0)),
        ],
        out_specs=pl.BlockSpec((ROWS_BLK, D_FEAT), lambda i: (i, 0)),
        out_shape=jax.ShapeDtypeStruct((N_NODES, D_FEAT), jnp.float32),
    )(degp, x)


def _tc_layer(degp, h, p, W1, b1, W2, b2, want_g):
    n_out = 3 if want_g else 2

    def body(degp_ref, h_ref, p_ref, w1_ref, b1_ref, w2_ref, b2_ref, *outs):
        dis = _dis_block(degp_ref[...])
        agg = (p_ref[0] + p_ref[1]) * dis
        hh = h_ref[...]
        t1 = jnp.dot(agg, w1_ref[...], preferred_element_type=jnp.float32) + b1_ref[...]
        t1 = jnp.where(t1 >= 0, t1, 0.2 * t1)
        t2 = jnp.dot(hh * agg, w2_ref[...], preferred_element_type=jnp.float32) + b2_ref[...]
        t2 = jnp.where(t2 >= 0, t2, 0.2 * t2)
        hnew = t1 + t2
        outs[0][...] = hnew
        sq = jnp.sum(hnew * hnew, axis=1, keepdims=True)
        outs[1][...] = hnew * lax.rsqrt(jnp.maximum(sq, 1e-12))
        if want_g:
            outs[2][...] = hnew * dis

    return pl.pallas_call(
        body,
        grid=(N_NODES // ROWS_BLK,),
        in_specs=[
            pl.BlockSpec((NC, ROWS_BLK, DEG_W), lambda i: (0, i, 0)),
            pl.BlockSpec((ROWS_BLK, D_FEAT), lambda i: (i, 0)),
            pl.BlockSpec((NC, ROWS_BLK, D_FEAT), lambda i: (0, i, 0)),
            pl.BlockSpec((D_FEAT, D_FEAT), lambda i: (0, 0)),
            pl.BlockSpec((1, D_FEAT), lambda i: (0, 0)),
            pl.BlockSpec((D_FEAT, D_FEAT), lambda i: (0, 0)),
            pl.BlockSpec((1, D_FEAT), lambda i: (0, 0)),
        ],
        out_specs=[pl.BlockSpec((ROWS_BLK, D_FEAT), lambda i: (i, 0))] * n_out,
        out_shape=[jax.ShapeDtypeStruct((N_NODES, D_FEAT), jnp.float32)] * n_out,
    )(degp, h, p, W1, b1, W2, b2)


def kernel(x, edge_index, W1_0, b1_0, W2_0, b2_0, W1_1, b1_1, W2_1, b2_1):
    row = edge_index[0]
    col = edge_index[1]
    b1_0r = b1_0.reshape(1, -1)
    b2_0r = b2_0.reshape(1, -1)
    b1_1r = b1_1.reshape(1, -1)
    b2_1r = b2_1.reshape(1, -1)

    degp = _sc_degree(row)
    g0 = _tc_prescale(degp, x)
    p1 = _sc_propagate(row, col, g0)
    h1, hn1, g1 = _tc_layer(degp, x, p1, W1_0, b1_0r, W2_0, b2_0r, want_g=True)
    p2 = _sc_propagate(row, col, g1)
    _, hn2 = _tc_layer(degp, h1, p2, W1_1, b1_1r, W2_1, b2_1r, want_g=False)
    return jnp.concatenate([x, hn1, hn2], axis=-1)


# speedup vs baseline: 9.5917x; 9.5917x over previous
import functools

import jax
import jax.numpy as jnp
from jax import lax
from jax.experimental import pallas as pl
from jax.experimental.pallas import tpu as pltpu
from jax.experimental.pallas import tpu_sc as plsc

N_NODES = 10000
N_PAD = 10240
D_FEAT = 128
N_EDGES = 320000

NC = 2
NS = 16
EPT = N_EDGES // (NC * NS)
CHUNK = 80
NCHUNK = EPT // CHUNK
RPT = N_PAD // NS
ZROWS = 128
DEG_W = 128

_MESH = dict(core_axis_name="c", subcore_axis_name="s")


def _sc_degree(row, ones_hbm_arr, zeros_hbm_arr):
    @functools.partial(
        pl.kernel,
        mesh=plsc.VectorSubcoreMesh(**_MESH),
        out_type=jax.ShapeDtypeStruct((NC, N_PAD, DEG_W), jnp.float32),
        scratch_types=[
            pltpu.VMEM((CHUNK,), jnp.int32),
            pltpu.VMEM((CHUNK, DEG_W), jnp.float32),
            pltpu.VMEM((ZROWS, DEG_W), jnp.float32),
            pltpu.VMEM_SHARED((N_PAD, DEG_W), jnp.float32),
        ],
    )
    def deg_kernel(row_hbm, ones_hbm, zeros_hbm, out_hbm, idx_v, ones_v, zbuf_v, hist_sh):
        cid = lax.axis_index("c")
        sid = lax.axis_index("s")
        pltpu.sync_copy(ones_hbm, ones_v)
        pltpu.sync_copy(zeros_hbm, zbuf_v)

        row0 = sid * RPT
        for z in range(RPT // ZROWS):
            pltpu.sync_copy(zbuf_v, hist_sh.at[pl.ds(row0 + z * ZROWS, ZROWS), :])
        plsc.subcore_barrier()

        ebase = (cid * NS + sid) * EPT

        def body(c, carry):
            base = pl.multiple_of(ebase + c * CHUNK, 8)
            pltpu.sync_copy(row_hbm.at[pl.ds(base, CHUNK)], idx_v)
            pltpu.sync_copy(ones_v, hist_sh.at[idx_v], add=True)
            return carry

        lax.fori_loop(0, NCHUNK, body, 0)
        plsc.subcore_barrier()
        pltpu.sync_copy(hist_sh.at[pl.ds(row0, RPT), :],
                        out_hbm.at[cid, pl.ds(row0, RPT), :])

    return deg_kernel(row, ones_hbm_arr, zeros_hbm_arr)


def _sc_propagate(row, col, g, zeros_hbm_arr):
    @functools.partial(
        pl.kernel,
        mesh=plsc.VectorSubcoreMesh(**_MESH),
        out_type=jax.ShapeDtypeStruct((NC, N_PAD, D_FEAT), jnp.float32),
        scratch_types=[
            pltpu.VMEM((CHUNK,), jnp.int32),
            pltpu.VMEM((CHUNK,), jnp.int32),
            pltpu.VMEM((CHUNK, D_FEAT), jnp.float32),
            pltpu.VMEM((ZROWS, D_FEAT), jnp.float32),
            pltpu.VMEM_SHARED((N_PAD, D_FEAT), jnp.float32),
            pltpu.SemaphoreType.DMA,
        ],
    )
    def gs_kernel(row_hbm, col_hbm, g_hbm, zeros_hbm, out_hbm,
                  cidx_v, ridx_v, rows_v, zbuf_v, agg_sh, sem):
        cid = lax.axis_index("c")
        sid = lax.axis_index("s")
        pltpu.sync_copy(zeros_hbm, zbuf_v)

        row0 = sid * RPT
        for z in range(RPT // ZROWS):
            pltpu.sync_copy(zbuf_v, agg_sh.at[pl.ds(row0 + z * ZROWS, ZROWS), :])
        plsc.subcore_barrier()

        ebase = (cid * NS + sid) * EPT

        def body(c, carry):
            base = pl.multiple_of(ebase + c * CHUNK, 8)
            pltpu.sync_copy(col_hbm.at[pl.ds(base, CHUNK)], cidx_v)
            pltpu.async_copy(g_hbm.at[cidx_v], rows_v, sem).wait()
            pltpu.sync_copy(row_hbm.at[pl.ds(base, CHUNK)], ridx_v)
            pltpu.sync_copy(rows_v, agg_sh.at[ridx_v], add=True)
            return carry

        lax.fori_loop(0, NCHUNK, body, 0)
        plsc.subcore_barrier()
        pltpu.sync_copy(agg_sh.at[pl.ds(row0, RPT), :],
                        out_hbm.at[cid, pl.ds(row0, RPT), :])

    return gs_kernel(row, col, g, zeros_hbm_arr)


ROWS_BLK = 1000


def _dis_block(degp):
    deg = (degp[0] + degp[1])[:, 0:1]
    return jnp.where(deg > 0, 1.0 / jnp.sqrt(deg), 0.0)


def _tc_prescale(degp, x):
    def body(degp_ref, x_ref, g_ref):
        g_ref[...] = x_ref[...] * _dis_block(degp_ref[...])

    return pl.pallas_call(
        body,
        grid=(N_NODES // ROWS_BLK,),
        in_specs=[
            pl.BlockSpec((NC, ROWS_BLK, DEG_W), lambda i: (0, i, 0)),
            pl.BlockSpec((ROWS_BLK, D_FEAT), lambda i: (i, 0)),
        ],
        out_specs=pl.BlockSpec((ROWS_BLK, D_FEAT), lambda i: (i, 0)),
        out_shape=jax.ShapeDtypeStruct((N_NODES, D_FEAT), jnp.float32),
    )(degp, x)


def _tc_layer(degp, h, p, W1, b1, W2, b2, want_g):
    n_out = 3 if want_g else 2

    def body(degp_ref, h_ref, p_ref, w1_ref, b1_ref, w2_ref, b2_ref, *outs):
        dis = _dis_block(degp_ref[...])
        agg = (p_ref[0] + p_ref[1]) * dis
        hh = h_ref[...]
        t1 = jnp.dot(agg, w1_ref[...], preferred_element_type=jnp.float32) + b1_ref[...]
        t1 = jnp.where(t1 >= 0, t1, 0.2 * t1)
        t2 = jnp.dot(hh * agg, w2_ref[...], preferred_element_type=jnp.float32) + b2_ref[...]
        t2 = jnp.where(t2 >= 0, t2, 0.2 * t2)
        hnew = t1 + t2
        outs[0][...] = hnew
        sq = jnp.sum(hnew * hnew, axis=1, keepdims=True)
        outs[1][...] = hnew * lax.rsqrt(jnp.maximum(sq, 1e-12))
        if want_g:
            outs[2][...] = hnew * dis

    return pl.pallas_call(
        body,
        grid=(N_NODES // ROWS_BLK,),
        in_specs=[
            pl.BlockSpec((NC, ROWS_BLK, DEG_W), lambda i: (0, i, 0)),
            pl.BlockSpec((ROWS_BLK, D_FEAT), lambda i: (i, 0)),
            pl.BlockSpec((NC, ROWS_BLK, D_FEAT), lambda i: (0, i, 0)),
            pl.BlockSpec((D_FEAT, D_FEAT), lambda i: (0, 0)),
            pl.BlockSpec((1, D_FEAT), lambda i: (0, 0)),
            pl.BlockSpec((D_FEAT, D_FEAT), lambda i: (0, 0)),
            pl.BlockSpec((1, D_FEAT), lambda i: (0, 0)),
        ],
        out_specs=[pl.BlockSpec((ROWS_BLK, D_FEAT), lambda i: (i, 0))] * n_out,
        out_shape=[jax.ShapeDtypeStruct((N_NODES, D_FEAT), jnp.float32)] * n_out,
    )(degp, h, p, W1, b1, W2, b2)


def kernel(x, edge_index, W1_0, b1_0, W2_0, b2_0, W1_1, b1_1, W2_1, b2_1):
    row = edge_index[0]
    col = edge_index[1]
    b1_0r = b1_0.reshape(1, -1)
    b2_0r = b2_0.reshape(1, -1)
    b1_1r = b1_1.reshape(1, -1)
    b2_1r = b2_1.reshape(1, -1)

    ones_c = jnp.ones((CHUNK, DEG_W), jnp.float32)
    zeros128 = jnp.zeros((ZROWS, D_FEAT), jnp.float32)

    degp = _sc_degree(row, ones_c, zeros128)
    g0 = _tc_prescale(degp, x)
    p1 = _sc_propagate(row, col, g0, zeros128)
    h1, hn1, g1 = _tc_layer(degp, x, p1, W1_0, b1_0r, W2_0, b2_0r, want_g=True)
    p2 = _sc_propagate(row, col, g1, zeros128)
    _, hn2 = _tc_layer(degp, h1, p2, W1_1, b1_1r, W2_1, b2_1r, want_g=False)
    return jnp.concatenate([x, hn1, hn2], axis=-1)


# speedup vs baseline: 20.4057x; 2.1274x over previous
import functools

import jax
import jax.numpy as jnp
from jax import lax
from jax.experimental import pallas as pl
from jax.experimental.pallas import tpu as pltpu
from jax.experimental.pallas import tpu_sc as plsc

N_NODES = 10000
N_PAD = 10240
D_FEAT = 128
N_EDGES = 320000

NC = 2
NS = 16
EPT = N_EDGES // (NC * NS)
CHUNK = 80
NCHUNK = EPT // CHUNK
RPT = N_PAD // NS
ZROWS = 64
NBUF = 5
IBUF = 4
RBUF = 2
DEG_W = 128

_MESH = dict(core_axis_name="c", subcore_axis_name="s")


def _sc_degree(row, ones_hbm_arr, zeros_hbm_arr):
    @functools.partial(
        pl.kernel,
        mesh=plsc.VectorSubcoreMesh(**_MESH),
        out_type=jax.ShapeDtypeStruct((NC, N_PAD, DEG_W), jnp.float32),
        scratch_types=(
            [pltpu.VMEM((CHUNK,), jnp.int32)] * NBUF
            + [
                pltpu.VMEM((CHUNK, DEG_W), jnp.float32),
                pltpu.VMEM((ZROWS, DEG_W), jnp.float32),
                pltpu.VMEM_SHARED((N_PAD, DEG_W), jnp.float32),
            ]
            + [pltpu.SemaphoreType.DMA] * NBUF
        ),
    )
    def deg_kernel(row_hbm, ones_hbm, zeros_hbm, out_hbm, *scr):
        idx_vs = scr[:NBUF]
        ones_v, zbuf_v, hist_sh = scr[NBUF:NBUF + 3]
        sems = scr[NBUF + 3:]
        cid = lax.axis_index("c")
        sid = lax.axis_index("s")
        pltpu.sync_copy(ones_hbm, ones_v)
        pltpu.sync_copy(zeros_hbm, zbuf_v)

        row0 = sid * RPT
        for z in range(RPT // ZROWS):
            pltpu.sync_copy(zbuf_v, hist_sh.at[pl.ds(row0 + z * ZROWS, ZROWS), :])
        plsc.subcore_barrier()

        ebase = (cid * NS + sid) * EPT

        def fire_idx(c, b):
            base = pl.multiple_of(ebase + c * CHUNK, 8)
            pltpu.async_copy(row_hbm.at[pl.ds(base, CHUNK)], idx_vs[b], sems[b])

        def wait_idx(c, b):
            base = pl.multiple_of(ebase + c * CHUNK, 8)
            pltpu.make_async_copy(row_hbm.at[pl.ds(base, CHUNK)], idx_vs[b],
                                  sems[b]).wait()

        fire_idx(0, 0)

        def outer(i, carry):
            for b in range(NBUF):
                c = i * NBUF + b
                nb = (b + 1) % NBUF

                @pl.when(c + 1 < NCHUNK)
                def _():
                    fire_idx(c + 1, nb)

                wait_idx(c, b)
                pltpu.sync_copy(ones_v, hist_sh.at[idx_vs[b]], add=True)
            return carry

        lax.fori_loop(0, NCHUNK // NBUF, outer, 0)
        plsc.subcore_barrier()
        pltpu.sync_copy(hist_sh.at[pl.ds(row0, RPT), :],
                        out_hbm.at[cid, pl.ds(row0, RPT), :])

    return deg_kernel(row, ones_hbm_arr, zeros_hbm_arr)


def _sc_propagate(row, col, g, zeros_hbm_arr):
    @functools.partial(
        pl.kernel,
        mesh=plsc.VectorSubcoreMesh(**_MESH),
        out_type=jax.ShapeDtypeStruct((NC, N_PAD, D_FEAT), jnp.float32),
        scratch_types=(
            [pltpu.VMEM((CHUNK,), jnp.int32)] * IBUF
            + [pltpu.VMEM((CHUNK,), jnp.int32)] * IBUF
            + [pltpu.VMEM((CHUNK, D_FEAT), jnp.float32)] * RBUF
            + [
                pltpu.VMEM((ZROWS, D_FEAT), jnp.float32),
                pltpu.VMEM_SHARED((N_PAD, D_FEAT), jnp.float32),
            ]
            + [pltpu.SemaphoreType.DMA] * IBUF
            + [pltpu.SemaphoreType.DMA] * RBUF
        ),
    )
    def gs_kernel(row_hbm, col_hbm, g_hbm, zeros_hbm, out_hbm, *scr):
        cidx_vs = scr[:IBUF]
        ridx_vs = scr[IBUF:2 * IBUF]
        rows_vs = scr[2 * IBUF:2 * IBUF + RBUF]
        zbuf_v, agg_sh = scr[2 * IBUF + RBUF:2 * IBUF + RBUF + 2]
        isems = scr[2 * IBUF + RBUF + 2:3 * IBUF + RBUF + 2]
        gsems = scr[3 * IBUF + RBUF + 2:]
        cid = lax.axis_index("c")
        sid = lax.axis_index("s")
        ebase = (cid * NS + sid) * EPT

        def fire_idx(c, ib):
            base = pl.multiple_of(ebase + c * CHUNK, 8)
            pltpu.async_copy(col_hbm.at[pl.ds(base, CHUNK)], cidx_vs[ib], isems[ib])
            pltpu.async_copy(row_hbm.at[pl.ds(base, CHUNK)], ridx_vs[ib], isems[ib])

        def wait_idx(c, ib):
            base = pl.multiple_of(ebase + c * CHUNK, 8)
            pltpu.make_async_copy(col_hbm.at[pl.ds(base, CHUNK)], cidx_vs[ib],
                                  isems[ib]).wait()
            pltpu.make_async_copy(row_hbm.at[pl.ds(base, CHUNK)], ridx_vs[ib],
                                  isems[ib]).wait()

        def fire_gather(ib, rb):
            pltpu.async_copy(g_hbm.at[cidx_vs[ib]], rows_vs[rb], gsems[rb])

        def wait_gather(ib, rb):
            pltpu.make_async_copy(g_hbm.at[cidx_vs[ib]], rows_vs[rb],
                                  gsems[rb]).wait()

        fire_idx(0, 0)
        pltpu.sync_copy(zeros_hbm, zbuf_v)
        wait_idx(0, 0)
        fire_gather(0, 0)
        fire_idx(1, 1)
        fire_idx(2, 2)

        row0 = sid * RPT
        for z in range(RPT // ZROWS):
            pltpu.sync_copy(zbuf_v, agg_sh.at[pl.ds(row0 + z * ZROWS, ZROWS), :])
        plsc.subcore_barrier()

        def do_chunk(c, b, last):
            if not last:
                wait_idx(c + 1, (b + 1) % IBUF)
                fire_gather((b + 1) % IBUF, (b + 1) % RBUF)

                @pl.when(c + 3 < NCHUNK)
                def _():
                    fire_idx(c + 3, (b + 3) % IBUF)

            wait_gather(b, b % RBUF)
            pltpu.sync_copy(rows_vs[b % RBUF], agg_sh.at[ridx_vs[b]], add=True)

        def outer(i, carry):
            for b in range(IBUF):
                do_chunk(i * IBUF + b, b, last=False)
            return carry

        lax.fori_loop(0, (NCHUNK - 1) // IBUF, outer, 0)
        do_chunk(NCHUNK - 1, (NCHUNK - 1) % IBUF, last=True)
        plsc.subcore_barrier()
        pltpu.sync_copy(agg_sh.at[pl.ds(row0, RPT), :],
                        out_hbm.at[cid, pl.ds(row0, RPT), :])

    return gs_kernel(row, col, g, zeros_hbm_arr)


ROWS_BLK = 1000


def _dis_block(degp):
    deg = (degp[0] + degp[1])[:, 0:1]
    return jnp.where(deg > 0, 1.0 / jnp.sqrt(deg), 0.0)


def _tc_prescale(degp, x):
    def body(degp_ref, x_ref, g_ref):
        g_ref[...] = x_ref[...] * _dis_block(degp_ref[...])

    return pl.pallas_call(
        body,
        grid=(N_NODES // ROWS_BLK,),
        in_specs=[
            pl.BlockSpec((NC, ROWS_BLK, DEG_W), lambda i: (0, i, 0)),
            pl.BlockSpec((ROWS_BLK, D_FEAT), lambda i: (i, 0)),
        ],
        out_specs=pl.BlockSpec((ROWS_BLK, D_FEAT), lambda i: (i, 0)),
        out_shape=jax.ShapeDtypeStruct((N_NODES, D_FEAT), jnp.float32),
    )(degp, x)


def _tc_layer(degp, h, p, W1, b1, W2, b2, want_g):
    n_out = 3 if want_g else 2

    def body(degp_ref, h_ref, p_ref, w1_ref, b1_ref, w2_ref, b2_ref, *outs):
        dis = _dis_block(degp_ref[...])
        agg = (p_ref[0] + p_ref[1]) * dis
        hh = h_ref[...]
        t1 = jnp.dot(agg, w1_ref[...], preferred_element_type=jnp.float32) + b1_ref[...]
        t1 = jnp.where(t1 >= 0, t1, 0.2 * t1)
        t2 = jnp.dot(hh * agg, w2_ref[...], preferred_element_type=jnp.float32) + b2_ref[...]
        t2 = jnp.where(t2 >= 0, t2, 0.2 * t2)
        hnew = t1 + t2
        outs[0][...] = hnew
        sq = jnp.sum(hnew * hnew, axis=1, keepdims=True)
        outs[1][...] = hnew * lax.rsqrt(jnp.maximum(sq, 1e-12))
        if want_g:
            outs[2][...] = hnew * dis

    return pl.pallas_call(
        body,
        grid=(N_NODES // ROWS_BLK,),
        in_specs=[
            pl.BlockSpec((NC, ROWS_BLK, DEG_W), lambda i: (0, i, 0)),
            pl.BlockSpec((ROWS_BLK, D_FEAT), lambda i: (i, 0)),
            pl.BlockSpec((NC, ROWS_BLK, D_FEAT), lambda i: (0, i, 0)),
            pl.BlockSpec((D_FEAT, D_FEAT), lambda i: (0, 0)),
            pl.BlockSpec((1, D_FEAT), lambda i: (0, 0)),
            pl.BlockSpec((D_FEAT, D_FEAT), lambda i: (0, 0)),
            pl.BlockSpec((1, D_FEAT), lambda i: (0, 0)),
        ],
        out_specs=[pl.BlockSpec((ROWS_BLK, D_FEAT), lambda i: (i, 0))] * n_out,
        out_shape=[jax.ShapeDtypeStruct((N_NODES, D_FEAT), jnp.float32)] * n_out,
    )(degp, h, p, W1, b1, W2, b2)


def kernel(x, edge_index, W1_0, b1_0, W2_0, b2_0, W1_1, b1_1, W2_1, b2_1):
    row = edge_index[0]
    col = edge_index[1]
    b1_0r = b1_0.reshape(1, -1)
    b2_0r = b2_0.reshape(1, -1)
    b1_1r = b1_1.reshape(1, -1)
    b2_1r = b2_1.reshape(1, -1)

    ones_c = jnp.ones((CHUNK, DEG_W), jnp.float32)
    zeros128 = jnp.zeros((ZROWS, D_FEAT), jnp.float32)

    degp = _sc_degree(row, ones_c, zeros128)
    g0 = _tc_prescale(degp, x)
    p1 = _sc_propagate(row, col, g0, zeros128)
    h1, hn1, g1 = _tc_layer(degp, x, p1, W1_0, b1_0r, W2_0, b2_0r, want_g=True)
    p2 = _sc_propagate(row, col, g1, zeros128)
    _, hn2 = _tc_layer(degp, h1, p2, W1_1, b1_1r, W2_1, b2_1r, want_g=False)
    return jnp.concatenate([x, hn1, hn2], axis=-1)


# speedup vs baseline: 23.7823x; 1.1655x over previous
import functools

import jax
import jax.numpy as jnp
from jax import lax
from jax.experimental import pallas as pl
from jax.experimental.pallas import tpu as pltpu
from jax.experimental.pallas import tpu_sc as plsc

N_NODES = 10000
N_PAD = 10240
D_FEAT = 128
N_EDGES = 320000

NC = 2
NS = 16
EPT = N_EDGES // (NC * NS)
CHUNK = 80
NCHUNK = EPT // CHUNK
RPT = N_PAD // NS
ZROWS = 64
NBUF = 5
IBUF = 4
RBUF = 2
DEG_W = 128

_MESH = dict(core_axis_name="c", subcore_axis_name="s")


def _sc_degree(row, zeros_hbm_arr):
    @functools.partial(
        pl.kernel,
        mesh=plsc.VectorSubcoreMesh(**_MESH),
        compiler_params=pltpu.CompilerParams(needs_layout_passes=False),
        out_type=jax.ShapeDtypeStruct((NC, NS, RPT), jnp.float32),
        scratch_types=[
            pltpu.VMEM((EPT,), jnp.int32),
            pltpu.VMEM((N_PAD,), jnp.float32),
            pltpu.VMEM((NS, RPT), jnp.float32),
            pltpu.VMEM((RPT,), jnp.float32),
            pltpu.VMEM_SHARED((NS, N_PAD), jnp.float32),
            pltpu.SemaphoreType.DMA,
        ],
    )
    def deg_kernel(row_hbm, zeros_hbm, out_hbm, idx_v, hist_v, mbuf_v, acc_v,
                   stage_sh, sem):
        cid = lax.axis_index("c")
        sid = lax.axis_index("s")
        ebase = (cid * NS + sid) * EPT
        pltpu.async_copy(row_hbm.at[pl.ds(pl.multiple_of(ebase, 8), EPT)],
                         idx_v, sem)
        pltpu.sync_copy(zeros_hbm, hist_v)
        pltpu.make_async_copy(row_hbm.at[pl.ds(pl.multiple_of(ebase, 8), EPT)],
                              idx_v, sem).wait()

        one16 = jnp.ones((16,), jnp.float32)

        def count(j, carry):
            rv = idx_v[pl.ds(j * 16, 16)]
            plsc.addupdate_scatter(hist_v, [rv], one16)
            return carry

        lax.fori_loop(0, EPT // 16, count, 0)
        pltpu.sync_copy(hist_v, stage_sh.at[sid])
        plsc.subcore_barrier()

        col0 = sid * RPT
        for k in range(NS):
            pltpu.sync_copy(stage_sh.at[k, pl.ds(col0, RPT)], mbuf_v.at[k])

        def merge(j, carry):
            s = mbuf_v[0, pl.ds(j * 16, 16)]
            for k in range(1, NS):
                s = s + mbuf_v[k, pl.ds(j * 16, 16)]
            acc_v[pl.ds(j * 16, 16)] = s
            return carry

        lax.fori_loop(0, RPT // 16, merge, 0)
        pltpu.sync_copy(acc_v, out_hbm.at[cid, sid])

    return deg_kernel(row, zeros_hbm_arr)


def _sc_propagate(row, col, g, zeros_hbm_arr):
    @functools.partial(
        pl.kernel,
        mesh=plsc.VectorSubcoreMesh(**_MESH),
        out_type=jax.ShapeDtypeStruct((NC, N_PAD, D_FEAT), jnp.float32),
        scratch_types=(
            [pltpu.VMEM((CHUNK,), jnp.int32)] * IBUF
            + [pltpu.VMEM((CHUNK,), jnp.int32)] * IBUF
            + [pltpu.VMEM((CHUNK, D_FEAT), jnp.float32)] * RBUF
            + [
                pltpu.VMEM((ZROWS, D_FEAT), jnp.float32),
                pltpu.VMEM_SHARED((N_PAD, D_FEAT), jnp.float32),
            ]
            + [pltpu.SemaphoreType.DMA] * IBUF
            + [pltpu.SemaphoreType.DMA] * RBUF
        ),
    )
    def gs_kernel(row_hbm, col_hbm, g_hbm, zeros_hbm, out_hbm, *scr):
        cidx_vs = scr[:IBUF]
        ridx_vs = scr[IBUF:2 * IBUF]
        rows_vs = scr[2 * IBUF:2 * IBUF + RBUF]
        zbuf_v, agg_sh = scr[2 * IBUF + RBUF:2 * IBUF + RBUF + 2]
        isems = scr[2 * IBUF + RBUF + 2:3 * IBUF + RBUF + 2]
        gsems = scr[3 * IBUF + RBUF + 2:]
        cid = lax.axis_index("c")
        sid = lax.axis_index("s")
        ebase = (cid * NS + sid) * EPT

        def fire_idx(c, ib):
            base = pl.multiple_of(ebase + c * CHUNK, 8)
            pltpu.async_copy(col_hbm.at[pl.ds(base, CHUNK)], cidx_vs[ib], isems[ib])
            pltpu.async_copy(row_hbm.at[pl.ds(base, CHUNK)], ridx_vs[ib], isems[ib])

        def wait_idx(c, ib):
            base = pl.multiple_of(ebase + c * CHUNK, 8)
            pltpu.make_async_copy(col_hbm.at[pl.ds(base, CHUNK)], cidx_vs[ib],
                                  isems[ib]).wait()
            pltpu.make_async_copy(row_hbm.at[pl.ds(base, CHUNK)], ridx_vs[ib],
                                  isems[ib]).wait()

        def fire_gather(ib, rb):
            pltpu.async_copy(g_hbm.at[cidx_vs[ib]], rows_vs[rb], gsems[rb])

        def wait_gather(ib, rb):
            pltpu.make_async_copy(g_hbm.at[cidx_vs[ib]], rows_vs[rb],
                                  gsems[rb]).wait()

        fire_idx(0, 0)
        pltpu.sync_copy(zeros_hbm, zbuf_v)
        wait_idx(0, 0)
        fire_gather(0, 0)
        fire_idx(1, 1)
        fire_idx(2, 2)

        row0 = sid * RPT
        for z in range(RPT // ZROWS):
            pltpu.sync_copy(zbuf_v, agg_sh.at[pl.ds(row0 + z * ZROWS, ZROWS), :])
        plsc.subcore_barrier()

        def do_chunk(c, b, last):
            if not last:
                wait_idx(c + 1, (b + 1) % IBUF)
                fire_gather((b + 1) % IBUF, (b + 1) % RBUF)

                @pl.when(c + 3 < NCHUNK)
                def _():
                    fire_idx(c + 3, (b + 3) % IBUF)

            wait_gather(b, b % RBUF)
            pltpu.sync_copy(rows_vs[b % RBUF], agg_sh.at[ridx_vs[b]], add=True)

        def outer(i, carry):
            for b in range(IBUF):
                do_chunk(i * IBUF + b, b, last=False)
            return carry

        lax.fori_loop(0, (NCHUNK - 1) // IBUF, outer, 0)
        do_chunk(NCHUNK - 1, (NCHUNK - 1) % IBUF, last=True)
        plsc.subcore_barrier()
        pltpu.sync_copy(agg_sh.at[pl.ds(row0, RPT), :],
                        out_hbm.at[cid, pl.ds(row0, RPT), :])

    return gs_kernel(row, col, g, zeros_hbm_arr)


ROWS_BLK = 1000


def _dis_block(deg):
    return jnp.where(deg > 0, 1.0 / jnp.sqrt(deg), 0.0)


def _tc_prescale(deg, x):
    def body(deg_ref, x_ref, g_ref):
        g_ref[...] = x_ref[...] * _dis_block(deg_ref[...])

    return pl.pallas_call(
        body,
        grid=(N_NODES // ROWS_BLK,),
        in_specs=[
            pl.BlockSpec((ROWS_BLK, 1), lambda i: (i, 0)),
            pl.BlockSpec((ROWS_BLK, D_FEAT), lambda i: (i, 0)),
        ],
        out_specs=pl.BlockSpec((ROWS_BLK, D_FEAT), lambda i: (i, 0)),
        out_shape=jax.ShapeDtypeStruct((N_NODES, D_FEAT), jnp.float32),
    )(deg, x)


def _tc_layer(deg, h, p, W1, b1, W2, b2, want_g):
    n_out = 3 if want_g else 2

    def body(deg_ref, h_ref, p_ref, w1_ref, b1_ref, w2_ref, b2_ref, *outs):
        dis = _dis_block(deg_ref[...])
        agg = (p_ref[0] + p_ref[1]) * dis
        hh = h_ref[...]
        t1 = jnp.dot(agg, w1_ref[...], preferred_element_type=jnp.float32) + b1_ref[...]
        t1 = jnp.where(t1 >= 0, t1, 0.2 * t1)
        t2 = jnp.dot(hh * agg, w2_ref[...], preferred_element_type=jnp.float32) + b2_ref[...]
        t2 = jnp.where(t2 >= 0, t2, 0.2 * t2)
        hnew = t1 + t2
        outs[0][...] = hnew
        sq = jnp.sum(hnew * hnew, axis=1, keepdims=True)
        outs[1][...] = hnew * lax.rsqrt(jnp.maximum(sq, 1e-12))
        if want_g:
            outs[2][...] = hnew * dis

    return pl.pallas_call(
        body,
        grid=(N_NODES // ROWS_BLK,),
        in_specs=[
            pl.BlockSpec((ROWS_BLK, 1), lambda i: (i, 0)),
            pl.BlockSpec((ROWS_BLK, D_FEAT), lambda i: (i, 0)),
            pl.BlockSpec((NC, ROWS_BLK, D_FEAT), lambda i: (0, i, 0)),
            pl.BlockSpec((D_FEAT, D_FEAT), lambda i: (0, 0)),
            pl.BlockSpec((1, D_FEAT), lambda i: (0, 0)),
            pl.BlockSpec((D_FEAT, D_FEAT), lambda i: (0, 0)),
            pl.BlockSpec((1, D_FEAT), lambda i: (0, 0)),
        ],
        out_specs=[pl.BlockSpec((ROWS_BLK, D_FEAT), lambda i: (i, 0))] * n_out,
        out_shape=[jax.ShapeDtypeStruct((N_NODES, D_FEAT), jnp.float32)] * n_out,
    )(deg, h, p, W1, b1, W2, b2)


def kernel(x, edge_index, W1_0, b1_0, W2_0, b2_0, W1_1, b1_1, W2_1, b2_1):
    row = edge_index[0]
    col = edge_index[1]
    b1_0r = b1_0.reshape(1, -1)
    b2_0r = b2_0.reshape(1, -1)
    b1_1r = b1_1.reshape(1, -1)
    b2_1r = b2_1.reshape(1, -1)

    zeros128 = jnp.zeros((ZROWS, D_FEAT), jnp.float32)
    zeros_pad = jnp.zeros((N_PAD,), jnp.float32)

    degp = _sc_degree(row, zeros_pad)
    deg = degp.sum(axis=0).reshape(N_PAD)[:N_NODES].reshape(N_NODES, 1)
    g0 = _tc_prescale(deg, x)
    p1 = _sc_propagate(row, col, g0, zeros128)
    h1, hn1, g1 = _tc_layer(deg, x, p1, W1_0, b1_0r, W2_0, b2_0r, want_g=True)
    p2 = _sc_propagate(row, col, g1, zeros128)
    _, hn2 = _tc_layer(deg, h1, p2, W1_1, b1_1r, W2_1, b2_1r, want_g=False)
    return jnp.concatenate([x, hn1, hn2], axis=-1)


# speedup vs baseline: 24.4444x; 1.0278x over previous
import functools

import jax
import jax.numpy as jnp
from jax import lax
from jax.experimental import pallas as pl
from jax.experimental.pallas import tpu as pltpu
from jax.experimental.pallas import tpu_sc as plsc

N_NODES = 10000
N_PAD = 10240
D_FEAT = 128
N_EDGES = 320000

NC = 2
NS = 16
EPT = N_EDGES // (NC * NS)
CHUNK = 80
NCHUNK = EPT // CHUNK
RPT = N_PAD // NS
ZROWS = 64
NBUF = 5
IBUF = 4
RBUF = 2
DEG_W = 128

_MESH = dict(core_axis_name="c", subcore_axis_name="s")


def _sc_degree(row, zeros_hbm_arr):
    @functools.partial(
        pl.kernel,
        mesh=plsc.VectorSubcoreMesh(**_MESH),
        compiler_params=pltpu.CompilerParams(needs_layout_passes=False),
        out_type=jax.ShapeDtypeStruct((NC, NS, RPT), jnp.float32),
        scratch_types=[
            pltpu.VMEM((EPT,), jnp.int32),
            pltpu.VMEM((N_PAD,), jnp.float32),
            pltpu.VMEM((NS, RPT), jnp.float32),
            pltpu.VMEM((RPT,), jnp.float32),
            pltpu.VMEM_SHARED((NS, N_PAD), jnp.float32),
            pltpu.SemaphoreType.DMA,
        ],
    )
    def deg_kernel(row_hbm, zeros_hbm, out_hbm, idx_v, hist_v, mbuf_v, acc_v,
                   stage_sh, sem):
        cid = lax.axis_index("c")
        sid = lax.axis_index("s")
        ebase = (cid * NS + sid) * EPT
        pltpu.async_copy(row_hbm.at[pl.ds(pl.multiple_of(ebase, 8), EPT)],
                         idx_v, sem)
        pltpu.sync_copy(zeros_hbm, hist_v)
        pltpu.make_async_copy(row_hbm.at[pl.ds(pl.multiple_of(ebase, 8), EPT)],
                              idx_v, sem).wait()

        one16 = jnp.ones((16,), jnp.float32)

        def count(j, carry):
            rv = idx_v[pl.ds(j * 16, 16)]
            plsc.addupdate_scatter(hist_v, [rv], one16)
            return carry

        lax.fori_loop(0, EPT // 16, count, 0)
        pltpu.sync_copy(hist_v, stage_sh.at[sid])
        plsc.subcore_barrier()

        col0 = sid * RPT
        for k in range(NS):
            pltpu.sync_copy(stage_sh.at[k, pl.ds(col0, RPT)], mbuf_v.at[k])

        def merge(j, carry):
            s = mbuf_v[0, pl.ds(j * 16, 16)]
            for k in range(1, NS):
                s = s + mbuf_v[k, pl.ds(j * 16, 16)]
            acc_v[pl.ds(j * 16, 16)] = s
            return carry

        lax.fori_loop(0, RPT // 16, merge, 0)
        pltpu.sync_copy(acc_v, out_hbm.at[cid, sid])

    return deg_kernel(row, zeros_hbm_arr)


def _sc_propagate(row, col, g, zeros_hbm_arr):
    @functools.partial(
        pl.kernel,
        mesh=plsc.VectorSubcoreMesh(**_MESH),
        out_type=jax.ShapeDtypeStruct((NC, N_PAD, D_FEAT), jnp.float32),
        scratch_types=(
            [pltpu.VMEM((CHUNK,), jnp.int32)] * IBUF
            + [pltpu.VMEM((CHUNK,), jnp.int32)] * IBUF
            + [pltpu.VMEM((CHUNK, D_FEAT), jnp.float32)] * RBUF
            + [
                pltpu.VMEM((ZROWS, D_FEAT), jnp.float32),
                pltpu.VMEM_SHARED((N_PAD, D_FEAT), jnp.float32),
            ]
            + [pltpu.SemaphoreType.DMA] * IBUF
            + [pltpu.SemaphoreType.DMA] * RBUF
        ),
    )
    def gs_kernel(row_hbm, col_hbm, g_hbm, zeros_hbm, out_hbm, *scr):
        cidx_vs = scr[:IBUF]
        ridx_vs = scr[IBUF:2 * IBUF]
        rows_vs = scr[2 * IBUF:2 * IBUF + RBUF]
        zbuf_v, agg_sh = scr[2 * IBUF + RBUF:2 * IBUF + RBUF + 2]
        isems = scr[2 * IBUF + RBUF + 2:3 * IBUF + RBUF + 2]
        gsems = scr[3 * IBUF + RBUF + 2:]
        cid = lax.axis_index("c")
        sid = lax.axis_index("s")
        ebase = (cid * NS + sid) * EPT

        def fire_idx(c, ib):
            base = pl.multiple_of(ebase + c * CHUNK, 8)
            pltpu.async_copy(col_hbm.at[pl.ds(base, CHUNK)], cidx_vs[ib], isems[ib])
            pltpu.async_copy(row_hbm.at[pl.ds(base, CHUNK)], ridx_vs[ib], isems[ib])

        def wait_idx(c, ib):
            base = pl.multiple_of(ebase + c * CHUNK, 8)
            pltpu.make_async_copy(col_hbm.at[pl.ds(base, CHUNK)], cidx_vs[ib],
                                  isems[ib]).wait()
            pltpu.make_async_copy(row_hbm.at[pl.ds(base, CHUNK)], ridx_vs[ib],
                                  isems[ib]).wait()

        def fire_gather(ib, rb):
            pltpu.async_copy(g_hbm.at[cidx_vs[ib]], rows_vs[rb], gsems[rb])

        def wait_gather(ib, rb):
            pltpu.make_async_copy(g_hbm.at[cidx_vs[ib]], rows_vs[rb],
                                  gsems[rb]).wait()

        fire_idx(0, 0)
        pltpu.sync_copy(zeros_hbm, zbuf_v)
        wait_idx(0, 0)
        fire_gather(0, 0)
        fire_idx(1, 1)
        fire_idx(2, 2)

        row0 = sid * RPT
        for z in range(RPT // ZROWS):
            pltpu.sync_copy(zbuf_v, agg_sh.at[pl.ds(row0 + z * ZROWS, ZROWS), :])
        plsc.subcore_barrier()

        def do_chunk(c, b, last):
            if not last:
                wait_idx(c + 1, (b + 1) % IBUF)
                fire_gather((b + 1) % IBUF, (b + 1) % RBUF)

                @pl.when(c + 3 < NCHUNK)
                def _():
                    fire_idx(c + 3, (b + 3) % IBUF)

            wait_gather(b, b % RBUF)
            pltpu.sync_copy(rows_vs[b % RBUF], agg_sh.at[ridx_vs[b]], add=True)

        def outer(i, carry):
            for b in range(IBUF):
                do_chunk(i * IBUF + b, b, last=False)
            return carry

        lax.fori_loop(0, (NCHUNK - 1) // IBUF, outer, 0)
        do_chunk(NCHUNK - 1, (NCHUNK - 1) % IBUF, last=True)
        plsc.subcore_barrier()
        pltpu.sync_copy(agg_sh.at[pl.ds(row0, RPT), :],
                        out_hbm.at[cid, pl.ds(row0, RPT), :])

    return gs_kernel(row, col, g, zeros_hbm_arr)


ROWS_BLK = 1000


def _dis_block(deg):
    return jnp.where(deg > 0, 1.0 / jnp.sqrt(deg), 0.0)


D_OUT = 3 * D_FEAT


def _tc_prescale(deg, x):
    def body(deg_ref, x_ref, g_ref, cat_ref):
        xx = x_ref[...]
        g_ref[...] = xx * _dis_block(deg_ref[...])
        cat_ref[...] = xx

    return pl.pallas_call(
        body,
        grid=(N_NODES // ROWS_BLK,),
        in_specs=[
            pl.BlockSpec((ROWS_BLK, 1), lambda i: (i, 0)),
            pl.BlockSpec((ROWS_BLK, D_FEAT), lambda i: (i, 0)),
        ],
        out_specs=[
            pl.BlockSpec((ROWS_BLK, D_FEAT), lambda i: (i, 0)),
            pl.BlockSpec((ROWS_BLK, D_FEAT), lambda i: (i, 0)),
        ],
        out_shape=[
            jax.ShapeDtypeStruct((N_NODES, D_FEAT), jnp.float32),
            jax.ShapeDtypeStruct((N_NODES, D_OUT), jnp.float32),
        ],
    )(deg, x)


def _tc_layer(deg, h, p, W1, b1, W2, b2, cat, strip, want_hg):
    def body(deg_ref, h_ref, p_ref, w1_ref, b1_ref, w2_ref, b2_ref, cat_in_ref,
             *outs):
        del cat_in_ref
        dis = _dis_block(deg_ref[...])
        agg = (p_ref[0] + p_ref[1]) * dis
        hh = h_ref[...]
        t1 = jnp.dot(agg, w1_ref[...], preferred_element_type=jnp.float32) + b1_ref[...]
        t1 = jnp.where(t1 >= 0, t1, 0.2 * t1)
        t2 = jnp.dot(hh * agg, w2_ref[...], preferred_element_type=jnp.float32) + b2_ref[...]
        t2 = jnp.where(t2 >= 0, t2, 0.2 * t2)
        hnew = t1 + t2
        sq = jnp.sum(hnew * hnew, axis=1, keepdims=True)
        outs[-1][...] = hnew * lax.rsqrt(jnp.maximum(sq, 1e-12))
        if want_hg:
            outs[0][...] = hnew
            outs[1][...] = hnew * dis

    n_extra = 2 if want_hg else 0
    outs = [jax.ShapeDtypeStruct((N_NODES, D_FEAT), jnp.float32)] * n_extra + [
        jax.ShapeDtypeStruct((N_NODES, D_OUT), jnp.float32)
    ]
    out_specs = [pl.BlockSpec((ROWS_BLK, D_FEAT), lambda i: (i, 0))] * n_extra + [
        pl.BlockSpec((ROWS_BLK, D_FEAT), lambda i, s=strip: (i, s))
    ]
    return pl.pallas_call(
        body,
        grid=(N_NODES // ROWS_BLK,),
        in_specs=[
            pl.BlockSpec((ROWS_BLK, 1), lambda i: (i, 0)),
            pl.BlockSpec((ROWS_BLK, D_FEAT), lambda i: (i, 0)),
            pl.BlockSpec((NC, ROWS_BLK, D_FEAT), lambda i: (0, i, 0)),
            pl.BlockSpec((D_FEAT, D_FEAT), lambda i: (0, 0)),
            pl.BlockSpec((1, D_FEAT), lambda i: (0, 0)),
            pl.BlockSpec((D_FEAT, D_FEAT), lambda i: (0, 0)),
            pl.BlockSpec((1, D_FEAT), lambda i: (0, 0)),
            pl.BlockSpec(memory_space=pl.ANY),
        ],
        out_specs=out_specs,
        out_shape=outs,
        input_output_aliases={7: n_extra},
    )(deg, h, p, W1, b1, W2, b2, cat)


def kernel(x, edge_index, W1_0, b1_0, W2_0, b2_0, W1_1, b1_1, W2_1, b2_1):
    row = edge_index[0]
    col = edge_index[1]
    b1_0r = b1_0.reshape(1, -1)
    b2_0r = b2_0.reshape(1, -1)
    b1_1r = b1_1.reshape(1, -1)
    b2_1r = b2_1.reshape(1, -1)

    zeros128 = jnp.zeros((ZROWS, D_FEAT), jnp.float32)
    zeros_pad = jnp.zeros((N_PAD,), jnp.float32)

    degp = _sc_degree(row, zeros_pad)
    deg = degp.sum(axis=0).reshape(N_PAD)[:N_NODES].reshape(N_NODES, 1)
    g0, cat0 = _tc_prescale(deg, x)
    p1 = _sc_propagate(row, col, g0, zeros128)
    h1, g1, cat1 = _tc_layer(deg, x, p1, W1_0, b1_0r, W2_0, b2_0r,
                             cat0, strip=1, want_hg=True)
    p2 = _sc_propagate(row, col, g1, zeros128)
    (cat2,) = _tc_layer(deg, h1, p2, W1_1, b1_1r, W2_1, b2_1r,
                        cat1, strip=2, want_hg=False)
    return cat2


# speedup vs baseline: 26.8552x; 1.0986x over previous
import functools

import jax
import jax.numpy as jnp
from jax import lax
from jax.experimental import pallas as pl
from jax.experimental.pallas import tpu as pltpu
from jax.experimental.pallas import tpu_sc as plsc

N_NODES = 10000
N_PAD = 10240
D_FEAT = 128
N_EDGES = 320000

NC = 2
NS = 16
EPT = N_EDGES // (NC * NS)
CHUNK = 128
NCHUNK_LO = 78
NCHUNK_HI = 79
XTRA = 4
RPT = N_PAD // NS
ZROWS = 32
IBUF = 4
RBUF = 2
DEG_W = 128

_MESH = dict(core_axis_name="c", subcore_axis_name="s")


def _sc_degree(row, zeros_hbm_arr):
    @functools.partial(
        pl.kernel,
        mesh=plsc.VectorSubcoreMesh(**_MESH),
        compiler_params=pltpu.CompilerParams(needs_layout_passes=False),
        out_type=jax.ShapeDtypeStruct((NC, NS, RPT), jnp.float32),
        scratch_types=[
            pltpu.VMEM((EPT,), jnp.int32),
            pltpu.VMEM((N_PAD,), jnp.float32),
            pltpu.VMEM((NS, RPT), jnp.float32),
            pltpu.VMEM((RPT,), jnp.float32),
            pltpu.VMEM_SHARED((NS, N_PAD), jnp.float32),
            pltpu.SemaphoreType.DMA,
        ],
    )
    def deg_kernel(row_hbm, zeros_hbm, out_hbm, idx_v, hist_v, mbuf_v, acc_v,
                   stage_sh, sem):
        cid = lax.axis_index("c")
        sid = lax.axis_index("s")
        ebase = (cid * NS + sid) * EPT
        pltpu.async_copy(row_hbm.at[pl.ds(pl.multiple_of(ebase, 8), EPT)],
                         idx_v, sem)
        pltpu.sync_copy(zeros_hbm, hist_v)
        pltpu.make_async_copy(row_hbm.at[pl.ds(pl.multiple_of(ebase, 8), EPT)],
                              idx_v, sem).wait()

        one16 = jnp.ones((16,), jnp.float32)

        def count(j, carry):
            rv = idx_v[pl.ds(j * 16, 16)]
            plsc.addupdate_scatter(hist_v, [rv], one16)
            return carry

        lax.fori_loop(0, EPT // 16, count, 0)
        pltpu.sync_copy(hist_v, stage_sh.at[sid])
        plsc.subcore_barrier()

        col0 = sid * RPT
        for k in range(NS):
            pltpu.sync_copy(stage_sh.at[k, pl.ds(col0, RPT)], mbuf_v.at[k])

        def merge(j, carry):
            s = mbuf_v[0, pl.ds(j * 16, 16)]
            for k in range(1, NS):
                s = s + mbuf_v[k, pl.ds(j * 16, 16)]
            acc_v[pl.ds(j * 16, 16)] = s
            return carry

        lax.fori_loop(0, RPT // 16, merge, 0)
        pltpu.sync_copy(acc_v, out_hbm.at[cid, sid])

    return deg_kernel(row, zeros_hbm_arr)


def _sc_propagate(row, col, g, zeros_hbm_arr):
    @functools.partial(
        pl.kernel,
        mesh=plsc.VectorSubcoreMesh(**_MESH),
        out_type=jax.ShapeDtypeStruct((NC, N_PAD, D_FEAT), jnp.float32),
        scratch_types=(
            [pltpu.VMEM((CHUNK,), jnp.int32)] * IBUF
            + [pltpu.VMEM((CHUNK,), jnp.int32)] * IBUF
            + [pltpu.VMEM((CHUNK, D_FEAT), jnp.float32)] * RBUF
            + [
                pltpu.VMEM((ZROWS, D_FEAT), jnp.float32),
                pltpu.VMEM_SHARED((N_PAD, D_FEAT), jnp.float32),
            ]
            + [pltpu.SemaphoreType.DMA] * IBUF
            + [pltpu.SemaphoreType.DMA] * RBUF
        ),
    )
    def gs_kernel(row_hbm, col_hbm, g_hbm, zeros_hbm, out_hbm, *scr):
        cidx_vs = scr[:IBUF]
        ridx_vs = scr[IBUF:2 * IBUF]
        rows_vs = scr[2 * IBUF:2 * IBUF + RBUF]
        zbuf_v, agg_sh = scr[2 * IBUF + RBUF:2 * IBUF + RBUF + 2]
        isems = scr[2 * IBUF + RBUF + 2:3 * IBUF + RBUF + 2]
        gsems = scr[3 * IBUF + RBUF + 2:]
        cid = lax.axis_index("c")
        sid = lax.axis_index("s")
        tid = cid * NS + sid
        ebase = jnp.where(
            tid < XTRA,
            tid * (NCHUNK_HI * CHUNK),
            XTRA * (NCHUNK_HI * CHUNK) + (tid - XTRA) * (NCHUNK_LO * CHUNK),
        )
        nch = jnp.where(tid < XTRA, NCHUNK_HI, NCHUNK_LO)

        def fire_idx(c, ib):
            base = pl.multiple_of(ebase + c * CHUNK, 8)
            pltpu.async_copy(col_hbm.at[pl.ds(base, CHUNK)], cidx_vs[ib], isems[ib])
            pltpu.async_copy(row_hbm.at[pl.ds(base, CHUNK)], ridx_vs[ib], isems[ib])

        def wait_idx(c, ib):
            base = pl.multiple_of(ebase + c * CHUNK, 8)
            pltpu.make_async_copy(col_hbm.at[pl.ds(base, CHUNK)], cidx_vs[ib],
                                  isems[ib]).wait()
            pltpu.make_async_copy(row_hbm.at[pl.ds(base, CHUNK)], ridx_vs[ib],
                                  isems[ib]).wait()

        def fire_gather(ib, rb):
            pltpu.async_copy(g_hbm.at[cidx_vs[ib]], rows_vs[rb], gsems[rb])

        def wait_gather(ib, rb):
            pltpu.make_async_copy(g_hbm.at[cidx_vs[ib]], rows_vs[rb],
                                  gsems[rb]).wait()

        fire_idx(0, 0)
        pltpu.sync_copy(zeros_hbm, zbuf_v)
        wait_idx(0, 0)
        fire_gather(0, 0)
        fire_idx(1, 1)
        fire_idx(2, 2)

        row0 = sid * RPT
        for z in range(RPT // ZROWS):
            pltpu.sync_copy(zbuf_v, agg_sh.at[pl.ds(row0 + z * ZROWS, ZROWS), :])
        plsc.subcore_barrier()

        def do_chunk(c, b):
            @pl.when(c + 1 < nch)
            def _():
                wait_idx(c + 1, (b + 1) % IBUF)
                fire_gather((b + 1) % IBUF, (b + 1) % RBUF)

            @pl.when(c + 3 < nch)
            def _():
                fire_idx(c + 3, (b + 3) % IBUF)

            wait_gather(b, b % RBUF)
            pltpu.sync_copy(rows_vs[b % RBUF], agg_sh.at[ridx_vs[b]], add=True)

        def outer(i, carry):
            for b in range(IBUF):
                do_chunk(i * IBUF + b, b)
            return carry

        lax.fori_loop(0, (NCHUNK_LO - 2) // IBUF, outer, 0)
        for c in range(((NCHUNK_LO - 2) // IBUF) * IBUF, NCHUNK_LO):
            do_chunk(c, c % IBUF)

        @pl.when(tid < XTRA)
        def _():
            do_chunk(NCHUNK_LO, NCHUNK_LO % IBUF)

        plsc.subcore_barrier()
        pltpu.sync_copy(agg_sh.at[pl.ds(row0, RPT), :],
                        out_hbm.at[cid, pl.ds(row0, RPT), :])

    return gs_kernel(row, col, g, zeros_hbm_arr)


ROWS_BLK = 2000


def _dis_block(deg):
    return jnp.where(deg > 0, 1.0 / jnp.sqrt(deg), 0.0)


D_OUT = 3 * D_FEAT


def _tc_prescale(deg, x):
    def body(deg_ref, x_ref, g_ref, cat_ref):
        xx = x_ref[...]
        g_ref[...] = xx * _dis_block(deg_ref[...])
        cat_ref[...] = xx

    return pl.pallas_call(
        body,
        grid=(N_NODES // ROWS_BLK,),
        in_specs=[
            pl.BlockSpec((ROWS_BLK, 1), lambda i: (i, 0)),
            pl.BlockSpec((ROWS_BLK, D_FEAT), lambda i: (i, 0)),
        ],
        out_specs=[
            pl.BlockSpec((ROWS_BLK, D_FEAT), lambda i: (i, 0)),
            pl.BlockSpec((ROWS_BLK, D_FEAT), lambda i: (i, 0)),
        ],
        out_shape=[
            jax.ShapeDtypeStruct((N_NODES, D_FEAT), jnp.float32),
            jax.ShapeDtypeStruct((N_NODES, D_OUT), jnp.float32),
        ],
    )(deg, x)


def _tc_layer(deg, h, p, W1, b1, W2, b2, cat, strip, want_hg):
    def body(deg_ref, h_ref, p_ref, w1_ref, b1_ref, w2_ref, b2_ref, cat_in_ref,
             *outs):
        del cat_in_ref
        dis = _dis_block(deg_ref[...])
        agg = (p_ref[0] + p_ref[1]) * dis
        hh = h_ref[...]
        t1 = jnp.dot(agg, w1_ref[...], preferred_element_type=jnp.float32) + b1_ref[...]
        t1 = jnp.where(t1 >= 0, t1, 0.2 * t1)
        t2 = jnp.dot(hh * agg, w2_ref[...], preferred_element_type=jnp.float32) + b2_ref[...]
        t2 = jnp.where(t2 >= 0, t2, 0.2 * t2)
        hnew = t1 + t2
        sq = jnp.sum(hnew * hnew, axis=1, keepdims=True)
        outs[-1][...] = hnew * lax.rsqrt(jnp.maximum(sq, 1e-12))
        if want_hg:
            outs[0][...] = hnew
            outs[1][...] = hnew * dis

    n_extra = 2 if want_hg else 0
    outs = [jax.ShapeDtypeStruct((N_NODES, D_FEAT), jnp.float32)] * n_extra + [
        jax.ShapeDtypeStruct((N_NODES, D_OUT), jnp.float32)
    ]
    out_specs = [pl.BlockSpec((ROWS_BLK, D_FEAT), lambda i: (i, 0))] * n_extra + [
        pl.BlockSpec((ROWS_BLK, D_FEAT), lambda i, s=strip: (i, s))
    ]
    return pl.pallas_call(
        body,
        grid=(N_NODES // ROWS_BLK,),
        in_specs=[
            pl.BlockSpec((ROWS_BLK, 1), lambda i: (i, 0)),
            pl.BlockSpec((ROWS_BLK, D_FEAT), lambda i: (i, 0)),
            pl.BlockSpec((NC, ROWS_BLK, D_FEAT), lambda i: (0, i, 0)),
            pl.BlockSpec((D_FEAT, D_FEAT), lambda i: (0, 0)),
            pl.BlockSpec((1, D_FEAT), lambda i: (0, 0)),
            pl.BlockSpec((D_FEAT, D_FEAT), lambda i: (0, 0)),
            pl.BlockSpec((1, D_FEAT), lambda i: (0, 0)),
            pl.BlockSpec(memory_space=pl.ANY),
        ],
        out_specs=out_specs,
        out_shape=outs,
        input_output_aliases={7: n_extra},
    )(deg, h, p, W1, b1, W2, b2, cat)


def kernel(x, edge_index, W1_0, b1_0, W2_0, b2_0, W1_1, b1_1, W2_1, b2_1):
    row = edge_index[0]
    col = edge_index[1]
    b1_0r = b1_0.reshape(1, -1)
    b2_0r = b2_0.reshape(1, -1)
    b1_1r = b1_1.reshape(1, -1)
    b2_1r = b2_1.reshape(1, -1)

    zeros128 = jnp.zeros((ZROWS, D_FEAT), jnp.float32)
    zeros_pad = jnp.zeros((N_PAD,), jnp.float32)

    degp = _sc_degree(row, zeros_pad)
    deg = degp.sum(axis=0).reshape(N_PAD)[:N_NODES].reshape(N_NODES, 1)
    g0, cat0 = _tc_prescale(deg, x)
    p1 = _sc_propagate(row, col, g0, zeros128)
    h1, g1, cat1 = _tc_layer(deg, x, p1, W1_0, b1_0r, W2_0, b2_0r,
                             cat0, strip=1, want_hg=True)
    p2 = _sc_propagate(row, col, g1, zeros128)
    (cat2,) = _tc_layer(deg, h1, p2, W1_1, b1_1r, W2_1, b2_1r,
                        cat1, strip=2, want_hg=False)
    return cat2


# speedup vs baseline: 26.9465x; 1.0034x over previous
import functools

import jax
import jax.numpy as jnp
from jax import lax
from jax.experimental import pallas as pl
from jax.experimental.pallas import tpu as pltpu
from jax.experimental.pallas import tpu_sc as plsc

N_NODES = 10000
N_PAD = 10240
D_FEAT = 128
N_EDGES = 320000

NC = 2
NS = 16
EPT = N_EDGES // (NC * NS)
CHUNK = 128
NCHUNK_LO = 78
NCHUNK_HI = 79
XTRA = 4
RPT = N_PAD // NS
ZROWS = 32
IBUF = 4
RBUF = 2
DEG_W = 128

_MESH = dict(core_axis_name="c", subcore_axis_name="s")


def _sc_degree(row, zeros_hbm_arr):
    @functools.partial(
        pl.kernel,
        mesh=plsc.VectorSubcoreMesh(**_MESH),
        compiler_params=pltpu.CompilerParams(needs_layout_passes=False),
        out_type=jax.ShapeDtypeStruct((NC, NS, RPT), jnp.float32),
        scratch_types=[
            pltpu.VMEM((EPT,), jnp.int32),
            pltpu.VMEM((N_PAD,), jnp.float32),
            pltpu.VMEM((NS, RPT), jnp.float32),
            pltpu.VMEM((RPT,), jnp.float32),
            pltpu.VMEM_SHARED((NS, N_PAD), jnp.float32),
            pltpu.SemaphoreType.DMA,
        ],
    )
    def deg_kernel(row_hbm, zeros_hbm, out_hbm, idx_v, hist_v, mbuf_v, acc_v,
                   stage_sh, sem):
        cid = lax.axis_index("c")
        sid = lax.axis_index("s")
        ebase = (cid * NS + sid) * EPT
        pltpu.async_copy(row_hbm.at[pl.ds(pl.multiple_of(ebase, 8), EPT)],
                         idx_v, sem)
        pltpu.sync_copy(zeros_hbm, hist_v)
        pltpu.make_async_copy(row_hbm.at[pl.ds(pl.multiple_of(ebase, 8), EPT)],
                              idx_v, sem).wait()

        one16 = jnp.ones((16,), jnp.float32)

        def count(j, carry):
            rv = idx_v[pl.ds(j * 16, 16)]
            plsc.addupdate_scatter(hist_v, [rv], one16)
            return carry

        lax.fori_loop(0, EPT // 16, count, 0)
        pltpu.sync_copy(hist_v, stage_sh.at[sid])
        plsc.subcore_barrier()

        col0 = sid * RPT
        for k in range(NS):
            pltpu.sync_copy(stage_sh.at[k, pl.ds(col0, RPT)], mbuf_v.at[k])

        def merge(j, carry):
            s = mbuf_v[0, pl.ds(j * 16, 16)]
            for k in range(1, NS):
                s = s + mbuf_v[k, pl.ds(j * 16, 16)]
            acc_v[pl.ds(j * 16, 16)] = s
            return carry

        lax.fori_loop(0, RPT // 16, merge, 0)
        pltpu.sync_copy(acc_v, out_hbm.at[cid, sid])

    return deg_kernel(row, zeros_hbm_arr)


def _sc_propagate(row, col, g, zeros_hbm_arr):
    @functools.partial(
        pl.kernel,
        mesh=plsc.VectorSubcoreMesh(**_MESH),
        out_type=jax.ShapeDtypeStruct((NC, N_PAD, D_FEAT), jnp.float32),
        scratch_types=(
            [pltpu.VMEM((CHUNK,), jnp.int32)] * IBUF
            + [pltpu.VMEM((CHUNK,), jnp.int32)] * IBUF
            + [pltpu.VMEM((CHUNK, D_FEAT), jnp.float32)] * RBUF
            + [
                pltpu.VMEM((ZROWS, D_FEAT), jnp.float32),
                pltpu.VMEM_SHARED((N_PAD, D_FEAT), jnp.float32),
            ]
            + [pltpu.SemaphoreType.DMA] * IBUF
            + [pltpu.SemaphoreType.DMA] * RBUF
        ),
    )
    def gs_kernel(row_hbm, col_hbm, g_hbm, zeros_hbm, out_hbm, *scr):
        cidx_vs = scr[:IBUF]
        ridx_vs = scr[IBUF:2 * IBUF]
        rows_vs = scr[2 * IBUF:2 * IBUF + RBUF]
        zbuf_v, agg_sh = scr[2 * IBUF + RBUF:2 * IBUF + RBUF + 2]
        isems = scr[2 * IBUF + RBUF + 2:3 * IBUF + RBUF + 2]
        gsems = scr[3 * IBUF + RBUF + 2:]
        cid = lax.axis_index("c")
        sid = lax.axis_index("s")
        tid = cid * NS + sid
        ebase = jnp.where(
            tid < XTRA,
            tid * (NCHUNK_HI * CHUNK),
            XTRA * (NCHUNK_HI * CHUNK) + (tid - XTRA) * (NCHUNK_LO * CHUNK),
        )
        nch = jnp.where(tid < XTRA, NCHUNK_HI, NCHUNK_LO)

        def fire_idx(c, ib):
            base = pl.multiple_of(ebase + c * CHUNK, 8)
            pltpu.async_copy(col_hbm.at[pl.ds(base, CHUNK)], cidx_vs[ib], isems[ib])
            pltpu.async_copy(row_hbm.at[pl.ds(base, CHUNK)], ridx_vs[ib], isems[ib])

        def wait_idx(c, ib):
            base = pl.multiple_of(ebase + c * CHUNK, 8)
            pltpu.make_async_copy(col_hbm.at[pl.ds(base, CHUNK)], cidx_vs[ib],
                                  isems[ib]).wait()
            pltpu.make_async_copy(row_hbm.at[pl.ds(base, CHUNK)], ridx_vs[ib],
                                  isems[ib]).wait()

        def fire_gather(ib, rb):
            pltpu.async_copy(g_hbm.at[cidx_vs[ib]], rows_vs[rb], gsems[rb])

        def wait_gather(ib, rb):
            pltpu.make_async_copy(g_hbm.at[cidx_vs[ib]], rows_vs[rb],
                                  gsems[rb]).wait()

        fire_idx(0, 0)
        pltpu.sync_copy(zeros_hbm, zbuf_v)
        wait_idx(0, 0)
        fire_gather(0, 0)
        fire_idx(1, 1)
        fire_idx(2, 2)

        row0 = sid * RPT
        for z in range(RPT // ZROWS):
            pltpu.sync_copy(zbuf_v, agg_sh.at[pl.ds(row0 + z * ZROWS, ZROWS), :])
        plsc.subcore_barrier()

        def do_chunk(c, b):
            @pl.when(c + 1 < nch)
            def _():
                wait_idx(c + 1, (b + 1) % IBUF)
                fire_gather((b + 1) % IBUF, (b + 1) % RBUF)

            @pl.when(c + 3 < nch)
            def _():
                fire_idx(c + 3, (b + 3) % IBUF)

            wait_gather(b, b % RBUF)
            pltpu.sync_copy(rows_vs[b % RBUF], agg_sh.at[ridx_vs[b]], add=True)

        def outer(i, carry):
            for b in range(IBUF):
                do_chunk(i * IBUF + b, b)
            return carry

        lax.fori_loop(0, (NCHUNK_LO - 2) // IBUF, outer, 0)
        for c in range(((NCHUNK_LO - 2) // IBUF) * IBUF, NCHUNK_LO):
            do_chunk(c, c % IBUF)

        @pl.when(tid < XTRA)
        def _():
            do_chunk(NCHUNK_LO, NCHUNK_LO % IBUF)

        plsc.subcore_barrier()
        pltpu.sync_copy(agg_sh.at[pl.ds(row0, RPT), :],
                        out_hbm.at[cid, pl.ds(row0, RPT), :])

    return gs_kernel(row, col, g, zeros_hbm_arr)


ROWS_BLK = 2000


def _dis_block(deg):
    return jnp.where(deg > 0, 1.0 / jnp.sqrt(deg), 0.0)


D_OUT = 3 * D_FEAT


def _tc_prescale(deg, x):
    def body(deg_ref, x_ref, g_ref):
        g_ref[...] = x_ref[...] * _dis_block(deg_ref[...])

    return pl.pallas_call(
        body,
        grid=(N_NODES // ROWS_BLK,),
        in_specs=[
            pl.BlockSpec((ROWS_BLK, 1), lambda i: (i, 0)),
            pl.BlockSpec((ROWS_BLK, D_FEAT), lambda i: (i, 0)),
        ],
        out_specs=pl.BlockSpec((ROWS_BLK, D_FEAT), lambda i: (i, 0)),
        out_shape=jax.ShapeDtypeStruct((N_NODES, D_FEAT), jnp.float32),
    )(deg, x)


def _tc_strip_write(src, cat, strip, normalize):
    def body(src_ref, *rest):
        s = src_ref[...]
        if normalize:
            sq = jnp.sum(s * s, axis=1, keepdims=True)
            s = s * lax.rsqrt(jnp.maximum(sq, 1e-12))
        rest[-1][...] = s

    in_specs = [pl.BlockSpec((ROWS_BLK, D_FEAT), lambda i: (i, 0))]
    operands = [src]
    aliases = {}
    if cat is not None:
        in_specs.append(pl.BlockSpec(memory_space=pl.ANY))
        operands.append(cat)
        aliases = {1: 0}
    return pl.pallas_call(
        body,
        grid=(N_NODES // ROWS_BLK,),
        in_specs=in_specs,
        out_specs=pl.BlockSpec((ROWS_BLK, D_FEAT), lambda i, s=strip: (i, s)),
        out_shape=jax.ShapeDtypeStruct((N_NODES, D_OUT), jnp.float32),
        input_output_aliases=aliases,
    )(*operands)


def _tc_layer(deg, h, p, W1, b1, W2, b2, cat, strip, want_hg):
    def body(deg_ref, h_ref, p_ref, w1_ref, b1_ref, w2_ref, b2_ref, *rest):
        dis = _dis_block(deg_ref[...])
        agg = (p_ref[0] + p_ref[1]) * dis
        hh = h_ref[...]
        t1 = jnp.dot(agg, w1_ref[...], preferred_element_type=jnp.float32) + b1_ref[...]
        t1 = jnp.where(t1 >= 0, t1, 0.2 * t1)
        t2 = jnp.dot(hh * agg, w2_ref[...], preferred_element_type=jnp.float32) + b2_ref[...]
        t2 = jnp.where(t2 >= 0, t2, 0.2 * t2)
        hnew = t1 + t2
        if want_hg:
            rest[0][...] = hnew
            rest[1][...] = hnew * dis
        else:
            sq = jnp.sum(hnew * hnew, axis=1, keepdims=True)
            rest[-1][...] = hnew * lax.rsqrt(jnp.maximum(sq, 1e-12))

    in_specs = [
        pl.BlockSpec((ROWS_BLK, 1), lambda i: (i, 0)),
        pl.BlockSpec((ROWS_BLK, D_FEAT), lambda i: (i, 0)),
        pl.BlockSpec((NC, ROWS_BLK, D_FEAT), lambda i: (0, i, 0)),
        pl.BlockSpec((D_FEAT, D_FEAT), lambda i: (0, 0)),
        pl.BlockSpec((1, D_FEAT), lambda i: (0, 0)),
        pl.BlockSpec((D_FEAT, D_FEAT), lambda i: (0, 0)),
        pl.BlockSpec((1, D_FEAT), lambda i: (0, 0)),
    ]
    operands = [deg, h, p, W1, b1, W2, b2]
    if want_hg:
        out_specs = [pl.BlockSpec((ROWS_BLK, D_FEAT), lambda i: (i, 0))] * 2
        outs = [jax.ShapeDtypeStruct((N_NODES, D_FEAT), jnp.float32)] * 2
        aliases = {}
    else:
        in_specs.append(pl.BlockSpec(memory_space=pl.ANY))
        operands.append(cat)
        out_specs = [pl.BlockSpec((ROWS_BLK, D_FEAT), lambda i, s=strip: (i, s))]
        outs = [jax.ShapeDtypeStruct((N_NODES, D_OUT), jnp.float32)]
        aliases = {7: 0}
    return pl.pallas_call(
        body,
        grid=(N_NODES // ROWS_BLK,),
        in_specs=in_specs,
        out_specs=out_specs,
        out_shape=outs,
        input_output_aliases=aliases,
    )(*operands)


def kernel(x, edge_index, W1_0, b1_0, W2_0, b2_0, W1_1, b1_1, W2_1, b2_1):
    row = edge_index[0]
    col = edge_index[1]
    b1_0r = b1_0.reshape(1, -1)
    b2_0r = b2_0.reshape(1, -1)
    b1_1r = b1_1.reshape(1, -1)
    b2_1r = b2_1.reshape(1, -1)

    zeros128 = jnp.zeros((ZROWS, D_FEAT), jnp.float32)
    zeros_pad = jnp.zeros((N_PAD,), jnp.float32)

    degp = _sc_degree(row, zeros_pad)
    deg = degp.sum(axis=0).reshape(N_PAD)[:N_NODES].reshape(N_NODES, 1)
    g0 = _tc_prescale(deg, x)
    cat0 = _tc_strip_write(x, None, 0, normalize=False)
    p1 = _sc_propagate(row, col, g0, zeros128)
    h1, g1 = _tc_layer(deg, x, p1, W1_0, b1_0r, W2_0, b2_0r,
                       None, strip=None, want_hg=True)
    p2 = _sc_propagate(row, col, g1, zeros128)
    cat1 = _tc_strip_write(h1, cat0, 1, normalize=True)
    (cat2,) = _tc_layer(deg, h1, p2, W1_1, b1_1r, W2_1, b2_1r,
                        cat1, strip=2, want_hg=False)
    return cat2
